# baseline (speedup 1.0000x reference)
import functools

import jax
import jax.numpy as jnp
from jax import lax
from jax.experimental import pallas as pl
from jax.experimental.pallas import tpu as pltpu
from jax.experimental.pallas import tpu_sc as plsc

BS = 512


def _tc_body(x_ref, emb_ref, out_ref):
    out_ref[...] = x_ref[...] + emb_ref[...][None, :, :]


def _tc_add(x, emb):
    B, T, D = x.shape
    return pl.pallas_call(
        _tc_body,
        grid=(T // BS,),
        in_specs=[
            pl.BlockSpec((B, BS, D), lambda i: (0, i, 0)),
            pl.BlockSpec((BS, D), lambda i: (i, 0)),
        ],
        out_specs=pl.BlockSpec((B, BS, D), lambda i: (0, i, 0)),
        out_shape=jax.ShapeDtypeStruct((B, T, D), x.dtype),
    )(x, emb)


_NC, _NS, _L = 2, 16, 16
_NW = _NC * _NS
_CH = 8


def _sc_add(x, emb):
    B, T, D = x.shape
    TW = T // _NW
    mesh = plsc.VectorSubcoreMesh(core_axis_name="c", subcore_axis_name="s")

    @functools.partial(
        pl.kernel,
        mesh=mesh,
        out_type=jax.ShapeDtypeStruct((B, T, D), jnp.float32),
        scratch_types=[
            pltpu.VMEM((_CH, D), jnp.float32),
            pltpu.VMEM((_CH, D), jnp.float32),
        ],
    )
    def k(x_hbm, emb_hbm, out_hbm, emb_v, x_v):
        wid = lax.axis_index("s") * _NC + lax.axis_index("c")
        t0 = wid * TW

        def chunk(c, carry):
            tc0 = t0 + c * _CH
            pltpu.sync_copy(emb_hbm.at[pl.ds(tc0, _CH)], emb_v)
            for b in range(B):
                pltpu.sync_copy(x_hbm.at[b, pl.ds(tc0, _CH)], x_v)

                def row(r, rc):
                    for j in range(D // _L):
                        sl = pl.ds(j * _L, _L)
                        x_v[r, sl] = x_v[r, sl] + emb_v[r, sl]
                    return rc

                lax.fori_loop(0, _CH, row, 0)
                pltpu.sync_copy(x_v, out_hbm.at[b, pl.ds(tc0, _CH)])
            return carry

        lax.fori_loop(0, TW // _CH, chunk, 0)

    return k(x, emb)


def kernel(x, embed):
    T = x.shape[1]
    return _sc_add(x, embed[:T])


# speedup vs baseline: 1.6995x; 1.6995x over previous
import functools

import jax
import jax.numpy as jnp
from jax import lax
from jax.experimental import pallas as pl
from jax.experimental.pallas import tpu as pltpu
from jax.experimental.pallas import tpu_sc as plsc

BS = 512


def _tc_body(x_ref, emb_ref, out_ref):
    out_ref[...] = x_ref[...] + emb_ref[...][None, :, :]


def _tc_add(x, emb):
    B, T, D = x.shape
    return pl.pallas_call(
        _tc_body,
        grid=(T // BS,),
        in_specs=[
            pl.BlockSpec((B, BS, D), lambda i: (0, i, 0)),
            pl.BlockSpec((BS, D), lambda i: (i, 0)),
        ],
        out_specs=pl.BlockSpec((B, BS, D), lambda i: (0, i, 0)),
        out_shape=jax.ShapeDtypeStruct((B, T, D), x.dtype),
    )(x, emb)


_NC, _NS, _L = 2, 16, 16
_NW = _NC * _NS
_CH = 8


def _sc_add(x, emb, t_lo, t_sc):
    B, T, D = x.shape
    TW = t_sc // _NW
    mesh = plsc.VectorSubcoreMesh(core_axis_name="c", subcore_axis_name="s")

    @functools.partial(
        pl.kernel,
        mesh=mesh,
        out_type=jax.ShapeDtypeStruct((B, t_sc, D), jnp.float32),
        scratch_types=[
            pltpu.VMEM((_CH, D), jnp.float32),
            pltpu.VMEM((_CH, D), jnp.float32),
        ],
    )
    def k(x_hbm, emb_hbm, out_hbm, emb_v, x_v):
        wid = lax.axis_index("s") * _NC + lax.axis_index("c")
        t0 = wid * TW

        def chunk(c, carry):
            tl0 = t0 + c * _CH
            pltpu.sync_copy(emb_hbm.at[pl.ds(t_lo + tl0, _CH)], emb_v)
            for b in range(B):
                pltpu.sync_copy(x_hbm.at[b, pl.ds(t_lo + tl0, _CH)], x_v)

                def row(r, rc):
                    for j in range(D // _L):
                        sl = pl.ds(j * _L, _L)
                        x_v[r, sl] = x_v[r, sl] + emb_v[r, sl]
                    return rc

                lax.fori_loop(0, _CH, row, 0)
                pltpu.sync_copy(x_v, out_hbm.at[b, pl.ds(tl0, _CH)])
            return carry

        lax.fori_loop(0, TW // _CH, chunk, 0)

    return k(x, emb)


T_SC = 512


def kernel(x, embed):
    B, T, D = x.shape
    emb = embed[:T]
    t_tc = T - T_SC
    sc_out = _sc_add(x, emb, t_tc, T_SC)
    tc_out = pl.pallas_call(
        _tc_body,
        grid=(t_tc // BS,),
        in_specs=[
            pl.BlockSpec((B, BS, D), lambda i: (0, i, 0)),
            pl.BlockSpec((BS, D), lambda i: (i, 0)),
        ],
        out_specs=pl.BlockSpec((B, BS, D), lambda i: (0, i, 0)),
        out_shape=jax.ShapeDtypeStruct((B, T, D), x.dtype),
    )(x, emb)
    return lax.dynamic_update_slice(tc_out, sc_out, (0, t_tc, 0))
